# baseline (speedup 1.0000x reference)
import jax
import jax.numpy as jnp
from jax import lax
from jax.experimental import pallas as pl
from jax.experimental.pallas import tpu as pltpu
from jax.experimental.pallas import tpu_sc as plsc

N_NODES = 10000
DEG = 32
D_FEAT = 128
NUM_LAYERS = 2

_NC = 2
_NS = 16
_NW = _NC * _NS

_G = 4
_E = _G * DEG
_CPW = 80
_NPAD = _NW * _CPW * _G
_NODES_PW = _CPW * _G
_LANES = 16
_NSLICE = D_FEAT // _LANES


def _compute_chunk(t, ws_v, rows_v, b, out_v):
    for g in range(_G):
        accs = [jnp.zeros((_LANES,), jnp.float32) for _ in range(_NSLICE)]
        for jg in range(DEG // _LANES):
            wv = ws_v[t, pl.ds(g * DEG + jg * _LANES, _LANES)]
            for j in range(_LANES):
                e = g * DEG + jg * _LANES + j
                w = wv[j]
                for k in range(_NSLICE):
                    accs[k] = accs[k] + (
                        rows_v[b, e, pl.ds(k * _LANES, _LANES)] * w)
        for k in range(_NSLICE):
            out_v[t * _G + g, pl.ds(k * _LANES, _LANES)] = accs[k]


def _layer_kernel(h_hbm, neigh_hbm, w_hbm, out_hbm, idxs_v, ws_v, rows_v,
                  out_v, gsem0, gsem1):
    wid = lax.axis_index("s") * _NC + lax.axis_index("c")
    pltpu.sync_copy(neigh_hbm.at[pl.ds(wid * _CPW, _CPW)], idxs_v)
    pltpu.sync_copy(w_hbm.at[pl.ds(wid * _CPW, _CPW)], ws_v)
    idxs2 = idxs_v
    ws2 = ws_v
    pltpu.async_copy(h_hbm.at[idxs2.at[0]], rows_v.at[0], gsem0)

    def body(tt, carry):
        for b, gsem_cur, gsem_nxt in ((0, gsem0, gsem1), (1, gsem1, gsem0)):
            t = tt * 2 + b
            tn = t + 1

            @pl.when(tn < _CPW)
            def _():
                pltpu.async_copy(h_hbm.at[idxs2.at[tn]], rows_v.at[1 - b],
                                 gsem_nxt)

            pltpu.make_async_copy(h_hbm.at[idxs2.at[t]], rows_v.at[b],
                                  gsem_cur).wait()
        return carry

    lax.fori_loop(0, _CPW // 2, body, 0)
    pltpu.sync_copy(out_v, out_hbm.at[pl.ds(wid * _NODES_PW, _NODES_PW)])


@jax.jit
def _run(raw_features, neigh_flat, w_flat):
    mesh = plsc.VectorSubcoreMesh(core_axis_name="c", subcore_axis_name="s")
    layer = pl.kernel(
        _layer_kernel,
        mesh=mesh,
        out_type=jax.ShapeDtypeStruct((_NPAD, D_FEAT), jnp.float32),
        scratch_types=[
            pltpu.VMEM((_CPW, _E), jnp.int32),
            pltpu.VMEM((_CPW, _E), jnp.float32),
            pltpu.VMEM((2, _E, D_FEAT), jnp.float32),
            pltpu.VMEM((_NODES_PW, D_FEAT), jnp.float32),
            pltpu.SemaphoreType.DMA,
            pltpu.SemaphoreType.DMA,
        ],
    )
    h = jnp.pad(raw_features, ((0, _NPAD - N_NODES), (0, 0)))
    for _ in range(NUM_LAYERS):
        h = layer(h, neigh_flat, w_flat)
    return h[:N_NODES]


def kernel(raw_features, neigh, weights, nodes_batch):
    del nodes_batch
    npad_e = _NPAD * DEG
    n_chunks = _NW * _CPW
    neigh_2d = jnp.pad(neigh.reshape(-1).astype(jnp.int32),
                       (0, npad_e - N_NODES * DEG)).reshape(n_chunks, _E)
    w_2d = jnp.pad(weights.reshape(-1).astype(jnp.float32),
                   (0, npad_e - N_NODES * DEG)).reshape(n_chunks, _E)
    return _run(raw_features, neigh_2d, w_2d)


# speedup vs baseline: 4.5246x; 4.5246x over previous
import jax
import jax.numpy as jnp
from jax import lax
from jax.experimental import pallas as pl
from jax.experimental.pallas import tpu as pltpu
from jax.experimental.pallas import tpu_sc as plsc

N_NODES = 10000
DEG = 32
D_FEAT = 128
NUM_LAYERS = 2

_NC = 2
_NS = 16
_NW = _NC * _NS

_G = 2
_E = _G * DEG
_CPW = 160
_NPAD = _NW * _CPW * _G
_NODES_PW = _CPW * _G
_LANES = 16
_NSLICE = D_FEAT // _LANES


def _compute_chunk(tt, b, ws_v, rows_v, out_v):
    for g in range(_G):
        accs = [jnp.zeros((_LANES,), jnp.float32) for _ in range(_NSLICE)]
        for jg in range(DEG // _LANES):
            wv = ws_v[tt, pl.ds(b * _E + g * DEG + jg * _LANES, _LANES)]
            for j in range(_LANES):
                e = g * DEG + jg * _LANES + j
                w = wv[j]
                for k in range(_NSLICE):
                    accs[k] = accs[k] + (
                        rows_v[b, e, pl.ds(k * _LANES, _LANES)] * w)
        for k in range(_NSLICE):
            out_v[b, g, pl.ds(k * _LANES, _LANES)] = accs[k]


def _layer_kernel(h_hbm, neigh_hbm, w_hbm, out_hbm, idxs_v, ws_v, rows_v,
                  out_v, h_sh, gsem0, gsem1, osem0, osem1):
    sid = lax.axis_index("s")
    wid = sid * _NC + lax.axis_index("c")
    stripe = _NPAD // _NS
    pltpu.sync_copy(h_hbm.at[pl.ds(sid * stripe, stripe)],
                    h_sh.at[pl.ds(sid * stripe, stripe)])
    pltpu.sync_copy(neigh_hbm.at[pl.ds(wid * (_CPW // 2), _CPW // 2)], idxs_v)
    pltpu.sync_copy(w_hbm.at[pl.ds(wid * (_CPW // 2), _CPW // 2)], ws_v)
    plsc.subcore_barrier()
    node0 = wid * _NODES_PW
    pltpu.async_copy(h_sh.at[idxs_v.at[0, pl.ds(0, _E)]], rows_v.at[0], gsem0)

    def body(tt, carry):
        for b, gsem_cur, gsem_nxt, osem_cur in (
                (0, gsem0, gsem1, osem0), (1, gsem1, gsem0, osem1)):
            lt = tt * 2 + b
            if b == 0:
                pltpu.async_copy(h_sh.at[idxs_v.at[tt, pl.ds(_E, _E)]],
                                 rows_v.at[1], gsem_nxt)
            else:
                @pl.when(tt + 1 < _CPW // 2)
                def _():
                    pltpu.async_copy(h_sh.at[idxs_v.at[tt + 1, pl.ds(0, _E)]],
                                     rows_v.at[0], gsem_nxt)

            pltpu.make_async_copy(h_sh.at[idxs_v.at[0, pl.ds(0, _E)]],
                                  rows_v.at[b], gsem_cur).wait()
            @pl.when(lt >= 2)
            def _():
                pltpu.make_async_copy(out_v.at[b],
                                      out_hbm.at[pl.ds(node0, _G)],
                                      osem_cur).wait()

            _compute_chunk(tt, b, ws_v, rows_v, out_v)
            pltpu.async_copy(out_v.at[b],
                             out_hbm.at[pl.ds(node0 + lt * _G, _G)], osem_cur)
        return carry

    lax.fori_loop(0, _CPW // 2, body, 0)
    pltpu.make_async_copy(out_v.at[0], out_hbm.at[pl.ds(node0, _G)],
                          osem0).wait()
    pltpu.make_async_copy(out_v.at[1], out_hbm.at[pl.ds(node0, _G)],
                          osem1).wait()


@jax.jit
def _run(raw_features, neigh_2d, w_2d):
    mesh = plsc.VectorSubcoreMesh(core_axis_name="c", subcore_axis_name="s")
    layer = pl.kernel(
        _layer_kernel,
        mesh=mesh,
        out_type=jax.ShapeDtypeStruct((_NPAD, D_FEAT), jnp.float32),
        scratch_types=[
            pltpu.VMEM((_CPW // 2, 2 * _E), jnp.int32),
            pltpu.VMEM((_CPW // 2, 2 * _E), jnp.float32),
            pltpu.VMEM((2, _E, D_FEAT), jnp.float32),
            pltpu.VMEM((2, _G, D_FEAT), jnp.float32),
            pltpu.VMEM_SHARED((_NPAD, D_FEAT), jnp.float32),
            pltpu.SemaphoreType.DMA,
            pltpu.SemaphoreType.DMA,
            pltpu.SemaphoreType.DMA,
            pltpu.SemaphoreType.DMA,
        ],
    )
    h = jnp.pad(raw_features, ((0, _NPAD - N_NODES), (0, 0)))
    for _ in range(NUM_LAYERS):
        h = layer(h, neigh_2d, w_2d)
    return h[:N_NODES]


def kernel(raw_features, neigh, weights, nodes_batch):
    del nodes_batch
    npad_e = _NPAD * DEG
    n_rows = _NW * (_CPW // 2)
    neigh_2d = jnp.pad(neigh.reshape(-1).astype(jnp.int32),
                       (0, npad_e - N_NODES * DEG)).reshape(n_rows, 2 * _E)
    w_2d = jnp.pad(weights.reshape(-1).astype(jnp.float32),
                   (0, npad_e - N_NODES * DEG)).reshape(n_rows, 2 * _E)
    return _run(raw_features, neigh_2d, w_2d)


# speedup vs baseline: 4.6512x; 1.0280x over previous
import jax
import jax.numpy as jnp
from jax import lax
from jax.experimental import pallas as pl
from jax.experimental.pallas import tpu as pltpu
from jax.experimental.pallas import tpu_sc as plsc

N_NODES = 10000
DEG = 32
D_FEAT = 128
NUM_LAYERS = 2

_NC = 2
_NS = 16
_NW = _NC * _NS

_G = 2
_E = _G * DEG
_CPW = 160
_NPAD = _NW * _CPW * _G
_NODES_PW = _CPW * _G
_LANES = 16
_NSLICE = D_FEAT // _LANES


def _compute_chunk(tt, b, ws_v, rows_v, out_v):
    for g in range(_G):
        for kh in range(2):
            accs = [jnp.zeros((_LANES,), jnp.float32)
                    for _ in range(_NSLICE // 2)]
            for jg in range(DEG // _LANES):
                wv = ws_v[tt, pl.ds(b * _E + g * DEG + jg * _LANES, _LANES)]
                for j in range(_LANES):
                    e = g * DEG + jg * _LANES + j
                    w = wv[j]
                    for k in range(_NSLICE // 2):
                        accs[k] = accs[k] + (
                            rows_v[b, e,
                                   pl.ds((kh * 4 + k) * _LANES, _LANES)] * w)
            for k in range(_NSLICE // 2):
                out_v[b, g, pl.ds((kh * 4 + k) * _LANES, _LANES)] = accs[k]


def _layer_kernel(h_hbm, neigh_hbm, w_hbm, out_hbm, idxs_v, ws_v, rows_v,
                  out_v, h_sh, gsem0, gsem1, osem0, osem1):
    sid = lax.axis_index("s")
    wid = sid * _NC + lax.axis_index("c")
    stripe = _NPAD // _NS
    pltpu.sync_copy(h_hbm.at[pl.ds(sid * stripe, stripe)],
                    h_sh.at[pl.ds(sid * stripe, stripe)])
    pltpu.sync_copy(neigh_hbm.at[pl.ds(wid * (_CPW // 2), _CPW // 2)], idxs_v)
    pltpu.sync_copy(w_hbm.at[pl.ds(wid * (_CPW // 2), _CPW // 2)], ws_v)
    plsc.subcore_barrier()
    node0 = wid * _NODES_PW
    pltpu.async_copy(h_sh.at[idxs_v.at[0, pl.ds(0, _E)]], rows_v.at[0], gsem0)

    def body(tt, carry):
        for b, gsem_cur, gsem_nxt, osem_cur in (
                (0, gsem0, gsem1, osem0), (1, gsem1, gsem0, osem1)):
            lt = tt * 2 + b
            if b == 0:
                pltpu.async_copy(h_sh.at[idxs_v.at[tt, pl.ds(_E, _E)]],
                                 rows_v.at[1], gsem_nxt)
            else:
                @pl.when(tt + 1 < _CPW // 2)
                def _():
                    pltpu.async_copy(h_sh.at[idxs_v.at[tt + 1, pl.ds(0, _E)]],
                                     rows_v.at[0], gsem_nxt)

            pltpu.make_async_copy(h_sh.at[idxs_v.at[0, pl.ds(0, _E)]],
                                  rows_v.at[b], gsem_cur).wait()
            @pl.when(lt >= 2)
            def _():
                pltpu.make_async_copy(out_v.at[b],
                                      out_hbm.at[pl.ds(node0, _G)],
                                      osem_cur).wait()

            _compute_chunk(tt, b, ws_v, rows_v, out_v)
            pltpu.async_copy(out_v.at[b],
                             out_hbm.at[pl.ds(node0 + lt * _G, _G)], osem_cur)
        return carry

    lax.fori_loop(0, _CPW // 2, body, 0)
    pltpu.make_async_copy(out_v.at[0], out_hbm.at[pl.ds(node0, _G)],
                          osem0).wait()
    pltpu.make_async_copy(out_v.at[1], out_hbm.at[pl.ds(node0, _G)],
                          osem1).wait()


@jax.jit
def _run(raw_features, neigh_2d, w_2d):
    mesh = plsc.VectorSubcoreMesh(core_axis_name="c", subcore_axis_name="s")
    layer = pl.kernel(
        _layer_kernel,
        mesh=mesh,
        out_type=jax.ShapeDtypeStruct((_NPAD, D_FEAT), jnp.float32),
        scratch_types=[
            pltpu.VMEM((_CPW // 2, 2 * _E), jnp.int32),
            pltpu.VMEM((_CPW // 2, 2 * _E), jnp.float32),
            pltpu.VMEM((2, _E, D_FEAT), jnp.float32),
            pltpu.VMEM((2, _G, D_FEAT), jnp.float32),
            pltpu.VMEM_SHARED((_NPAD, D_FEAT), jnp.float32),
            pltpu.SemaphoreType.DMA,
            pltpu.SemaphoreType.DMA,
            pltpu.SemaphoreType.DMA,
            pltpu.SemaphoreType.DMA,
        ],
    )
    h = jnp.pad(raw_features, ((0, _NPAD - N_NODES), (0, 0)))
    for _ in range(NUM_LAYERS):
        h = layer(h, neigh_2d, w_2d)
    return h[:N_NODES]


def kernel(raw_features, neigh, weights, nodes_batch):
    del nodes_batch
    npad_e = _NPAD * DEG
    n_rows = _NW * (_CPW // 2)
    neigh_2d = jnp.pad(neigh.reshape(-1).astype(jnp.int32),
                       (0, npad_e - N_NODES * DEG)).reshape(n_rows, 2 * _E)
    w_2d = jnp.pad(weights.reshape(-1).astype(jnp.float32),
                   (0, npad_e - N_NODES * DEG)).reshape(n_rows, 2 * _E)
    return _run(raw_features, neigh_2d, w_2d)
